# Initial kernel scaffold; baseline (speedup 1.0000x reference)
import jax
import jax.numpy as jnp
from jax.experimental import pallas as pl


def kernel(token_ids, weight):
    raise NotImplementedError("write your pallas kernel here")


# speedup vs baseline: 3.1253x; 3.1253x over previous
import functools

import jax
import jax.numpy as jnp
from jax import lax
from jax.experimental import pallas as pl
from jax.experimental.pallas import tpu as pltpu
from jax.experimental.pallas import tpu_sc as plsc

_EMBED_DIM = 128
_CHUNK = 128


@functools.lru_cache(maxsize=None)
def _build(num_tokens: int, dim: int):
    info = plsc.get_sparse_core_info()
    nw = info.num_cores * info.num_subcores
    b_per_w = num_tokens // nw
    assert b_per_w % _CHUNK == 0
    n_chunks = b_per_w // _CHUNK
    mesh = plsc.VectorSubcoreMesh(core_axis_name="c", subcore_axis_name="s")

    @functools.partial(
        pl.kernel,
        mesh=mesh,
        out_type=jax.ShapeDtypeStruct((num_tokens, dim), jnp.float32),
        scratch_types=[
            pltpu.VMEM((n_chunks, _CHUNK), jnp.int32),
            pltpu.VMEM((2, _CHUNK, dim), jnp.float32),
            pltpu.SemaphoreType.DMA,
            pltpu.SemaphoreType.DMA,
        ],
    )
    def gather_kernel(idx_hbm, table_hbm, out_hbm, idx_v, rows_v, gsem, ssem):
        wid = lax.axis_index("s") * info.num_cores + lax.axis_index("c")
        base = wid * b_per_w
        pltpu.sync_copy(idx_hbm.at[wid], idx_v)

        pltpu.async_copy(table_hbm.at[idx_v.at[0]], rows_v.at[0], gsem)

        def body(i, _):
            slot = lax.rem(i, 2)
            nxt = lax.rem(i + 1, 2)
            pltpu.make_async_copy(
                table_hbm.at[idx_v.at[i]], rows_v.at[slot], gsem
            ).wait()

            @pl.when(i > 0)
            def _():
                pltpu.make_async_copy(
                    rows_v.at[nxt], out_hbm.at[pl.ds(base, _CHUNK)], ssem
                ).wait()

            @pl.when(i + 1 < n_chunks)
            def _():
                pltpu.async_copy(
                    table_hbm.at[idx_v.at[i + 1]], rows_v.at[nxt], gsem
                )

            pltpu.async_copy(
                rows_v.at[slot], out_hbm.at[pl.ds(base + i * _CHUNK, _CHUNK)], ssem
            )
            return 0

        lax.fori_loop(0, n_chunks, body, 0)
        pltpu.make_async_copy(
            rows_v.at[0], out_hbm.at[pl.ds(base, _CHUNK)], ssem
        ).wait()

    return gather_kernel


def kernel(token_ids, weight):
    b, s = token_ids.shape
    num_tokens = b * s
    info = plsc.get_sparse_core_info()
    nw = info.num_cores * info.num_subcores
    idx = jnp.asarray(token_ids, jnp.int32).reshape(
        nw, num_tokens // (nw * _CHUNK), _CHUNK
    )
    out = _build(num_tokens, weight.shape[1])(idx, weight)
    return out.reshape(b, s, weight.shape[1])


# speedup vs baseline: 3.3545x; 1.0734x over previous
import functools

import jax
import jax.numpy as jnp
from jax import lax
from jax.experimental import pallas as pl
from jax.experimental.pallas import tpu as pltpu
from jax.experimental.pallas import tpu_sc as plsc

_EMBED_DIM = 128
_CHUNK = 128


@functools.lru_cache(maxsize=None)
def _build(num_tokens: int, dim: int):
    info = plsc.get_sparse_core_info()
    nw = info.num_cores * info.num_subcores
    b_per_w = num_tokens // nw
    assert b_per_w % _CHUNK == 0
    n_chunks = b_per_w // _CHUNK
    mesh = plsc.VectorSubcoreMesh(core_axis_name="c", subcore_axis_name="s")

    nbuf = 4
    assert n_chunks >= nbuf

    @functools.partial(
        pl.kernel,
        mesh=mesh,
        out_type=jax.ShapeDtypeStruct((num_tokens, dim), jnp.float32),
        scratch_types=[
            pltpu.VMEM((n_chunks, _CHUNK), jnp.int32),
            pltpu.VMEM((nbuf, _CHUNK, dim), jnp.float32),
            pltpu.SemaphoreType.DMA,
            pltpu.SemaphoreType.DMA,
        ],
    )
    def gather_kernel(idx_hbm, table_hbm, out_hbm, idx_v, rows_v, gsem, ssem):
        wid = lax.axis_index("s") * info.num_cores + lax.axis_index("c")
        base = wid * b_per_w
        pltpu.sync_copy(idx_hbm.at[wid], idx_v)

        for b in range(nbuf - 1):
            pltpu.async_copy(table_hbm.at[idx_v.at[b]], rows_v.at[b], gsem)

        def body(i, _):
            slot = lax.rem(i, nbuf)
            pltpu.make_async_copy(
                table_hbm.at[idx_v.at[i]], rows_v.at[slot], gsem
            ).wait()
            pltpu.async_copy(
                rows_v.at[slot], out_hbm.at[pl.ds(base + i * _CHUNK, _CHUNK)], ssem
            )
            j = i + nbuf - 1

            @pl.when(jnp.logical_and(i >= 1, j < n_chunks))
            def _():
                pltpu.make_async_copy(
                    rows_v.at[0], out_hbm.at[pl.ds(base, _CHUNK)], ssem
                ).wait()

            @pl.when(j < n_chunks)
            def _():
                pltpu.async_copy(
                    table_hbm.at[idx_v.at[j]], rows_v.at[lax.rem(j, nbuf)], gsem
                )

            return 0

        lax.fori_loop(0, n_chunks, body, 0)
        for _ in range(nbuf):
            pltpu.make_async_copy(
                rows_v.at[0], out_hbm.at[pl.ds(base, _CHUNK)], ssem
            ).wait()

    return gather_kernel


def kernel(token_ids, weight):
    b, s = token_ids.shape
    num_tokens = b * s
    info = plsc.get_sparse_core_info()
    nw = info.num_cores * info.num_subcores
    idx = jnp.asarray(token_ids, jnp.int32).reshape(
        nw, num_tokens // (nw * _CHUNK), _CHUNK
    )
    out = _build(num_tokens, weight.shape[1])(idx, weight)
    return out.reshape(b, s, weight.shape[1])


# speedup vs baseline: 3.3668x; 1.0037x over previous
import functools

import jax
import jax.numpy as jnp
from jax import lax
from jax.experimental import pallas as pl
from jax.experimental.pallas import tpu as pltpu
from jax.experimental.pallas import tpu_sc as plsc

_EMBED_DIM = 128
_IDXW = 128
_K = 1
_NBUF = 6


@functools.lru_cache(maxsize=None)
def _build(num_tokens: int, dim: int):
    info = plsc.get_sparse_core_info()
    nw = info.num_cores * info.num_subcores
    b_per_w = num_tokens // nw
    idx_rows = b_per_w // _IDXW
    n_chunks = idx_rows // _K
    assert n_chunks * _K == idx_rows and n_chunks >= _NBUF
    mesh = plsc.VectorSubcoreMesh(core_axis_name="c", subcore_axis_name="s")

    @functools.partial(
        pl.kernel,
        mesh=mesh,
        out_type=jax.ShapeDtypeStruct((num_tokens, dim), jnp.float32),
        scratch_types=[
            pltpu.VMEM((idx_rows, _IDXW), jnp.int32),
            pltpu.VMEM((_NBUF, _IDXW, dim), jnp.float32),
            pltpu.SemaphoreType.DMA,
            pltpu.SemaphoreType.DMA,
        ],
    )
    def gather_kernel(idx_hbm, table_hbm, out_hbm, idx_v, rows_v, gsem, ssem):
        wid = lax.axis_index("s") * info.num_cores + lax.axis_index("c")
        base = wid * idx_rows
        pltpu.sync_copy(idx_hbm.at[wid], idx_v)

        for b in range(_NBUF - 1):
            pltpu.async_copy(
                table_hbm.at[idx_v.at[b]], rows_v.at[b], gsem
            )

        def body(i, _):
            slot = lax.rem(i, _NBUF)
            pltpu.make_async_copy(
                table_hbm.at[idx_v.at[i]], rows_v.at[slot], gsem
            ).wait()
            pltpu.async_copy(
                rows_v.at[slot], out_hbm.at[pl.ds((base + i) * _IDXW, _IDXW)], ssem
            )
            j = i + _NBUF - 1

            @pl.when(jnp.logical_and(i >= 1, j < n_chunks))
            def _():
                pltpu.make_async_copy(
                    rows_v.at[0], out_hbm.at[pl.ds(base * _IDXW, _IDXW)], ssem
                ).wait()

            @pl.when(j < n_chunks)
            def _():
                pltpu.async_copy(
                    table_hbm.at[idx_v.at[j]],
                    rows_v.at[lax.rem(j, _NBUF)],
                    gsem,
                )

            return 0

        lax.fori_loop(0, n_chunks, body, 0)
        for _ in range(_NBUF):
            pltpu.make_async_copy(
                rows_v.at[0], out_hbm.at[pl.ds(base * _IDXW, _IDXW)], ssem
            ).wait()

    return gather_kernel


def kernel(token_ids, weight):
    b, s = token_ids.shape
    num_tokens = b * s
    info = plsc.get_sparse_core_info()
    nw = info.num_cores * info.num_subcores
    idx = jnp.asarray(token_ids, jnp.int32).reshape(
        nw, num_tokens // (nw * _IDXW), _IDXW
    )
    out = _build(num_tokens, weight.shape[1])(idx, weight)
    return out.reshape(b, s, weight.shape[1])


# speedup vs baseline: 6.0018x; 1.7826x over previous
import functools

import jax
import jax.numpy as jnp
from jax import lax
from jax.experimental import pallas as pl
from jax.experimental.pallas import tpu as pltpu
from jax.experimental.pallas import tpu_sc as plsc

_NBUF = 8


@functools.lru_cache(maxsize=None)
def _build(n_seq: int, seq_len: int, dim: int):
    info = plsc.get_sparse_core_info()
    nw = info.num_cores * info.num_subcores
    s_per_w = n_seq // nw
    assert s_per_w * nw == n_seq and s_per_w >= _NBUF and seq_len <= 128
    mesh = plsc.VectorSubcoreMesh(core_axis_name="c", subcore_axis_name="s")

    @functools.partial(
        pl.kernel,
        mesh=mesh,
        out_type=jax.ShapeDtypeStruct((n_seq, seq_len, dim), jnp.float32),
        scratch_types=[
            pltpu.VMEM((s_per_w, seq_len), jnp.int32),
            pltpu.VMEM((_NBUF, seq_len, dim), jnp.float32),
            pltpu.SemaphoreType.DMA,
            pltpu.SemaphoreType.DMA,
        ],
    )
    def gather_kernel(idx_hbm, table_hbm, out_hbm, idx_v, rows_v, gsem, ssem):
        wid = lax.axis_index("s") * info.num_cores + lax.axis_index("c")
        base = wid * s_per_w
        pltpu.sync_copy(idx_hbm.at[wid], idx_v)

        for b in range(_NBUF - 1):
            pltpu.async_copy(table_hbm.at[idx_v.at[b]], rows_v.at[b], gsem)

        def body(i, _):
            slot = lax.rem(i, _NBUF)
            pltpu.make_async_copy(
                table_hbm.at[idx_v.at[i]], rows_v.at[slot], gsem
            ).wait()
            pltpu.async_copy(rows_v.at[slot], out_hbm.at[base + i], ssem)
            j = i + _NBUF - 1

            @pl.when(jnp.logical_and(i >= 1, j < s_per_w))
            def _():
                pltpu.make_async_copy(
                    rows_v.at[0], out_hbm.at[base], ssem
                ).wait()

            @pl.when(j < s_per_w)
            def _():
                pltpu.async_copy(
                    table_hbm.at[idx_v.at[j]], rows_v.at[lax.rem(j, _NBUF)], gsem
                )

            return 0

        lax.fori_loop(0, s_per_w, body, 0)
        for _ in range(_NBUF):
            pltpu.make_async_copy(rows_v.at[0], out_hbm.at[base], ssem).wait()

    return gather_kernel


def kernel(token_ids, weight):
    b, s = token_ids.shape
    info = plsc.get_sparse_core_info()
    nw = info.num_cores * info.num_subcores
    idx = jnp.asarray(token_ids, jnp.int32).reshape(nw, b // nw, s)
    return _build(b, s, weight.shape[1])(idx, weight)
